# baseline (speedup 1.0000x reference)
import functools

import jax
import jax.numpy as jnp
from jax import lax
from jax.experimental import pallas as pl
from jax.experimental.pallas import tpu as pltpu
from jax.experimental.pallas import tpu_sc as plsc

_K = 16
_NEG = jnp.float32(-3e38)


def _scores_body(xr_ref, xa_ref, o_ref):
    xr = xr_ref[0]
    xa = xa_ref[0]
    xx = jnp.sum(xa * xa, axis=1)
    o_ref[0] = 2.0 * jax.lax.dot_general(
        xr, xa, (((1,), (1,)), ((), ())),
        preferred_element_type=jnp.float32) - xx[None, :]


def _tc_scores(x):
    B, N, C = x.shape
    R = 256
    return pl.pallas_call(
        _scores_body,
        grid=(B, N // R),
        in_specs=[
            pl.BlockSpec((1, R, C), lambda b, i: (b, i, 0)),
            pl.BlockSpec((1, N, C), lambda b, i: (b, 0, 0)),
        ],
        out_specs=pl.BlockSpec((1, R, N), lambda b, i: (b, i, 0)),
        out_shape=jax.ShapeDtypeStruct((B, N, N), jnp.float32),
    )(x, x)


def _dense_body(h_ref, wl_ref, bl_ref, wc_ref, bc_ref, o_ref):
    h = h_ref[0]
    w = jax.lax.dot_general(wl_ref[...], wc_ref[...],
                            (((1,), (0,)), ((), ())),
                            preferred_element_type=jnp.float32)
    bias = jax.lax.dot_general(bl_ref[...], wc_ref[...],
                               (((1,), (0,)), ((), ())),
                               preferred_element_type=jnp.float32) + bc_ref[...]
    out = jax.lax.dot_general(h, w, (((1,), (0,)), ((), ())),
                              preferred_element_type=jnp.float32) + bias
    o_ref[0] = jnp.maximum(out, 0.0)


def _tc_dense(h, W_lin, b_lin, W_conv, b_conv):
    B, N, C = h.shape
    out_f = W_conv.shape[1]
    bl = b_lin.reshape(1, -1)
    bc = b_conv.reshape(1, -1)
    return pl.pallas_call(
        _dense_body,
        grid=(B,),
        in_specs=[
            pl.BlockSpec((1, N, C), lambda b: (b, 0, 0)),
            pl.BlockSpec((C, W_lin.shape[1]), lambda b: (0, 0)),
            pl.BlockSpec((1, b_lin.shape[0]), lambda b: (0, 0)),
            pl.BlockSpec((W_conv.shape[0], out_f), lambda b: (0, 0)),
            pl.BlockSpec((1, out_f), lambda b: (0, 0)),
        ],
        out_specs=pl.BlockSpec((1, N, out_f), lambda b: (b, 0, 0)),
        out_shape=jax.ShapeDtypeStruct((B, N, out_f), jnp.float32),
    )(h, W_lin, bl, W_conv, bc)


def _make_sc_topk(BN, N, C):
    NC, NS = 2, 16
    NW = NC * NS
    rows_w = BN // NW
    nchunks = N // 16
    mesh = plsc.VectorSubcoreMesh(core_axis_name="c", subcore_axis_name="s")

    @functools.partial(
        pl.kernel, mesh=mesh,
        compiler_params=pltpu.CompilerParams(needs_layout_passes=False),
        out_type=jax.ShapeDtypeStruct((BN * C,), jnp.float32),
        scratch_types=[
            pltpu.VMEM((N,), jnp.float32),
            pltpu.VMEM((16,), jnp.float32),
            pltpu.VMEM((16,), jnp.int32),
            pltpu.VMEM((16,), jnp.int32),
            pltpu.VMEM((16, 128), jnp.float32),
            pltpu.VMEM((rows_w * C,), jnp.float32),
            pltpu.SemaphoreType.DMA,
        ],
    )
    def sc_topk(scores_hbm, xflat_hbm, out_hbm,
                sbuf, tbuf, ibuf, idxg, rows_v, hbuf, sem):
        wid = lax.axis_index("s") * NC + lax.axis_index("c")
        base = wid * rows_w
        boff = (base // N) * N

        def row_body(r, carry):
            row = base + r
            pltpu.sync_copy(scores_hbm.at[row], sbuf)
            tbuf[...] = jnp.full((16,), _NEG, jnp.float32)
            ibuf[...] = jnp.zeros((16,), jnp.int32)

            def chunk_body(j, carry2):
                v = sbuf[pl.ds(j * 16, 16)]
                thr = tbuf[...][0]
                cnt = plsc.all_reduce_population_count(v > thr)
                hit = cnt[0] > 0

                @pl.when(hit)
                def _():
                    idx = j * 16 + lax.iota(jnp.int32, 16)
                    vd, idd = plsc.sort_key_val(v, idx, descending=True)
                    t = tbuf[...]
                    i = ibuf[...]
                    take = vd > t
                    t2 = jnp.where(take, vd, t)
                    i2 = jnp.where(take, idd, i)
                    ts, is_ = plsc.sort_key_val(t2, i2, descending=False)
                    tbuf[...] = ts
                    ibuf[...] = is_

                return carry2

            lax.fori_loop(0, nchunks, chunk_body, 0)

            idxg[...] = ibuf[...] + boff
            pltpu.async_copy(xflat_hbm.at[idxg], rows_v, sem).wait()
            h = rows_v[0, pl.ds(0, C)]
            for i in range(1, 16):
                h = jnp.maximum(h, rows_v[i, pl.ds(0, C)])
            hbuf[pl.ds(r * C, C)] = h
            return carry

        lax.fori_loop(0, rows_w, row_body, 0)
        pltpu.sync_copy(hbuf, out_hbm.at[pl.ds(base * C, rows_w * C)])

    return sc_topk


@functools.partial(jax.jit, static_argnames=())
def kernel(x, W_lin, b_lin, W_conv, b_conv):
    B, N, C = x.shape
    scores = _tc_scores(x).reshape(B * N, N)
    xflat = jnp.pad(x.reshape(B * N, C), ((0, 0), (0, 128 - C)))
    h = _make_sc_topk(B * N, N, C)(scores, xflat).reshape(B, N, C)
    return _tc_dense(h, W_lin, b_lin, W_conv, b_conv)


# speedup vs baseline: 2.0184x; 2.0184x over previous
import functools

import jax
import jax.numpy as jnp
from jax import lax
from jax.experimental import pallas as pl
from jax.experimental.pallas import tpu as pltpu
from jax.experimental.pallas import tpu_sc as plsc

_K = 16
_GB = 8
_NEG = jnp.float32(-3e38)


def _scores_body(xr_ref, xa_ref, o_ref):
    xr = xr_ref[0]
    xa = xa_ref[0]
    xx = jnp.sum(xa * xa, axis=1)
    o_ref[0] = 2.0 * jax.lax.dot_general(
        xr, xa, (((1,), (1,)), ((), ())),
        preferred_element_type=jnp.float32) - xx[None, :]


def _tc_scores(x):
    B, N, C = x.shape
    R = 256
    return pl.pallas_call(
        _scores_body,
        grid=(B, N // R),
        in_specs=[
            pl.BlockSpec((1, R, C), lambda b, i: (b, i, 0)),
            pl.BlockSpec((1, N, C), lambda b, i: (b, 0, 0)),
        ],
        out_specs=pl.BlockSpec((1, R, N), lambda b, i: (b, i, 0)),
        out_shape=jax.ShapeDtypeStruct((B, N, N), jnp.float32),
    )(x, x)


def _dense_body(h_ref, wl_ref, bl_ref, wc_ref, bc_ref, o_ref):
    h = h_ref[0]
    w = jax.lax.dot_general(wl_ref[...], wc_ref[...],
                            (((1,), (0,)), ((), ())),
                            preferred_element_type=jnp.float32)
    bias = jax.lax.dot_general(bl_ref[...], wc_ref[...],
                               (((1,), (0,)), ((), ())),
                               preferred_element_type=jnp.float32) + bc_ref[...]
    out = jax.lax.dot_general(h, w, (((1,), (0,)), ((), ())),
                              preferred_element_type=jnp.float32) + bias
    o_ref[0] = jnp.maximum(out, 0.0)


def _tc_dense(h, W_lin, b_lin, W_conv, b_conv):
    B, N, C = h.shape
    out_f = W_conv.shape[1]
    bl = b_lin.reshape(1, -1)
    bc = b_conv.reshape(1, -1)
    return pl.pallas_call(
        _dense_body,
        grid=(B,),
        in_specs=[
            pl.BlockSpec((1, N, C), lambda b: (b, 0, 0)),
            pl.BlockSpec((C, W_lin.shape[1]), lambda b: (0, 0)),
            pl.BlockSpec((1, b_lin.shape[0]), lambda b: (0, 0)),
            pl.BlockSpec((W_conv.shape[0], out_f), lambda b: (0, 0)),
            pl.BlockSpec((1, out_f), lambda b: (0, 0)),
        ],
        out_specs=pl.BlockSpec((1, N, out_f), lambda b: (b, 0, 0)),
        out_shape=jax.ShapeDtypeStruct((B, N, out_f), jnp.float32),
    )(h, W_lin, bl, W_conv, bc)


def _make_sc_topk(BN, N, C):
    NC, NS = 2, 16
    NW = NC * NS
    rows_w = BN // NW
    nchunks = N // 16
    mesh = plsc.VectorSubcoreMesh(core_axis_name="c", subcore_axis_name="s")

    @functools.partial(
        pl.kernel, mesh=mesh,
        compiler_params=pltpu.CompilerParams(needs_layout_passes=False),
        out_type=jax.ShapeDtypeStruct((BN * C,), jnp.float32),
        scratch_types=[
            pltpu.VMEM((2 * N,), jnp.float32),
            pltpu.VMEM((16,), jnp.float32),
            pltpu.VMEM((16,), jnp.int32),
            pltpu.VMEM((rows_w * 16,), jnp.int32),
            pltpu.VMEM((_GB * 16, 128), jnp.float32),
            pltpu.VMEM((rows_w * C,), jnp.float32),
            pltpu.SemaphoreType.DMA,
            pltpu.SemaphoreType.DMA,
            pltpu.SemaphoreType.DMA,
        ],
    )
    def sc_topk(scores_hbm, xflat_hbm, out_hbm,
                sbuf, tbuf, ibuf, idxa, rows_v, hbuf, sem, sem1, gsem):
        wid = lax.axis_index("s") * NC + lax.axis_index("c")
        base = wid * rows_w
        boff = (base // N) * N

        pltpu.async_copy(scores_hbm.at[base], sbuf.at[pl.ds(0, N)], sem).wait()

        def select_row(r, buf):
            tbuf[...] = jnp.full((16,), _NEG, jnp.float32)

            def chunk_body(j, carry2):
                j0 = j * 4
                v0 = buf[pl.ds(j0 * 16, 16)]
                v1 = buf[pl.ds((j0 + 1) * 16, 16)]
                v2 = buf[pl.ds((j0 + 2) * 16, 16)]
                v3 = buf[pl.ds((j0 + 3) * 16, 16)]
                thr = tbuf[...][0]
                any4 = ((v0 > thr) | (v1 > thr)) | ((v2 > thr) | (v3 > thr))
                cnt = plsc.all_reduce_population_count(any4)
                hit = cnt[0] > 0

                @pl.when(hit)
                def _():
                    for q, v in enumerate((v0, v1, v2, v3)):
                        idx = (j0 + q) * 16 + lax.iota(jnp.int32, 16)
                        vd, idd = plsc.sort_key_val(v, idx, descending=True)
                        t = tbuf[...]
                        i = ibuf[...]
                        take = vd > t
                        t2 = jnp.where(take, vd, t)
                        i2 = jnp.where(take, idd, i)
                        ts, is_ = plsc.sort_key_val(t2, i2, descending=False)
                        tbuf[...] = ts
                        ibuf[...] = is_

                return carry2

            lax.fori_loop(0, nchunks // 4, chunk_body, 0)
            idxa[pl.ds(r * 16, 16)] = ibuf[...] + boff

        def pair_body(p, carry):
            r0 = 2 * p
            pltpu.async_copy(scores_hbm.at[base + r0 + 1], sbuf.at[pl.ds(N, N)], sem1)
            select_row(r0, sbuf.at[pl.ds(0, N)])

            @pl.when(r0 + 2 < rows_w)
            def _():
                pltpu.async_copy(scores_hbm.at[base + r0 + 2], sbuf.at[pl.ds(0, N)], sem)

            pltpu.make_async_copy(scores_hbm.at[base + r0 + 1],
                                  sbuf.at[pl.ds(N, N)], sem1).wait()
            select_row(r0 + 1, sbuf.at[pl.ds(N, N)])

            @pl.when(r0 + 2 < rows_w)
            def _():
                pltpu.make_async_copy(scores_hbm.at[base + r0 + 2],
                                      sbuf.at[pl.ds(0, N)], sem).wait()

            return carry

        lax.fori_loop(0, rows_w // 2, pair_body, 0)

        def gather_body(g, carry):
            pltpu.async_copy(
                xflat_hbm.at[idxa.at[pl.ds(g * _GB * 16, _GB * 16)]],
                rows_v, gsem).wait()
            for rr in range(_GB):
                h = rows_v[rr * 16, pl.ds(0, C)]
                for i in range(1, 16):
                    h = jnp.maximum(h, rows_v[rr * 16 + i, pl.ds(0, C)])
                hbuf[pl.ds((g * _GB + rr) * C, C)] = h
            return carry

        lax.fori_loop(0, rows_w // _GB, gather_body, 0)
        pltpu.sync_copy(hbuf, out_hbm.at[pl.ds(base * C, rows_w * C)])

    return sc_topk


@functools.partial(jax.jit, static_argnames=())
def kernel(x, W_lin, b_lin, W_conv, b_conv):
    B, N, C = x.shape
    scores = _tc_scores(x).reshape(B * N, N)
    xflat = jnp.pad(x.reshape(B * N, C), ((0, 0), (0, 128 - C)))
    h = _make_sc_topk(B * N, N, C)(scores, xflat).reshape(B, N, C)
    return _tc_dense(h, W_lin, b_lin, W_conv, b_conv)


# speedup vs baseline: 4.7205x; 2.3388x over previous
import functools

import jax
import jax.numpy as jnp
from jax.experimental import pallas as pl

_K = 16


def _body(xr_ref, xa_ref, wl_ref, bl_ref, wc_ref, bc_ref, o_ref):
    xr = xr_ref[0]
    xa = xa_ref[0]

    xx = jnp.sum(xa * xa, axis=1)
    st = 2.0 * jax.lax.dot_general(
        xa, xr, (((1,), (1,)), ((), ())),
        preferred_element_type=jnp.float32) - xx[:, None]

    iota = jax.lax.broadcasted_iota(jnp.int32, st.shape, 0)
    neg = jnp.float32(-1e30)
    r0 = pl.program_id(1) * xr.shape[0]
    rowi = jax.lax.broadcasted_iota(jnp.int32, st.shape, 1) + r0
    st = jnp.where(iota == rowi, neg, st)
    h = xr
    for _ in range(_K - 1):
        sel = jnp.argmax(st, axis=0).astype(jnp.int32)[None, :]
        onehot = (iota == sel)
        g = jax.lax.dot_general(
            onehot.astype(jnp.float32), xa, (((0,), (0,)), ((), ())),
            preferred_element_type=jnp.float32)
        h = jnp.maximum(h, g)
        st = jnp.where(onehot, neg, st)

    w = jax.lax.dot_general(wl_ref[...], wc_ref[...],
                            (((1,), (0,)), ((), ())),
                            preferred_element_type=jnp.float32)
    bias = jax.lax.dot_general(bl_ref[...], wc_ref[...],
                               (((1,), (0,)), ((), ())),
                               preferred_element_type=jnp.float32) + bc_ref[...]
    out = jax.lax.dot_general(h, w, (((1,), (0,)), ((), ())),
                              preferred_element_type=jnp.float32) + bias
    o_ref[0] = jnp.maximum(out, 0.0)


@functools.partial(jax.jit, static_argnames=())
def kernel(x, W_lin, b_lin, W_conv, b_conv):
    B, N, C = x.shape
    R = 256
    out_f = W_conv.shape[1]
    bl = b_lin.reshape(1, -1)
    bc = b_conv.reshape(1, -1)
    grid = (B, N // R)
    return pl.pallas_call(
        _body,
        grid=grid,
        in_specs=[
            pl.BlockSpec((1, R, C), lambda b, i: (b, i, 0)),
            pl.BlockSpec((1, N, C), lambda b, i: (b, 0, 0)),
            pl.BlockSpec((C, W_lin.shape[1]), lambda b, i: (0, 0)),
            pl.BlockSpec((1, b_lin.shape[0]), lambda b, i: (0, 0)),
            pl.BlockSpec((W_conv.shape[0], out_f), lambda b, i: (0, 0)),
            pl.BlockSpec((1, out_f), lambda b, i: (0, 0)),
        ],
        out_specs=pl.BlockSpec((1, R, out_f), lambda b, i: (b, i, 0)),
        out_shape=jax.ShapeDtypeStruct((B, N, out_f), jnp.float32),
    )(x, x, W_lin, bl, W_conv, bc)
